# baseline (speedup 1.0000x reference)
import functools

import jax
import jax.numpy as jnp
from jax import lax
from jax.experimental import pallas as pl
from jax.experimental.pallas import tpu as pltpu
from jax.experimental.pallas import tpu_sc as plsc

NUM_CORES = 2
NUM_SUBCORES = 16
NW = NUM_CORES * NUM_SUBCORES
LANES = 16


def _make_sc_pool(batch, seq, emb_dim, cb, gb):
    rows_per_w = batch // NW
    chunks_per_w = rows_per_w // cb
    tok_per_chunk = cb * seq
    nb = tok_per_chunk // gb
    assert nb * gb == tok_per_chunk
    dgroups = emb_dim // LANES
    inv = 1.0 / float(seq)

    mesh = plsc.VectorSubcoreMesh(
        core_axis_name="c", subcore_axis_name="s",
        num_cores=NUM_CORES, num_subcores=NUM_SUBCORES)

    @functools.partial(
        pl.kernel,
        out_type=jax.ShapeDtypeStruct((batch, emb_dim), jnp.float32),
        mesh=mesh,
        scratch_types=[
            pltpu.VMEM((nb, gb), jnp.int32),
            pltpu.VMEM((tok_per_chunk, emb_dim), jnp.float32),
            pltpu.VMEM((cb, emb_dim), jnp.float32),
            pltpu.SemaphoreType.DMA,
        ],
        compiler_params=pltpu.CompilerParams(use_tc_tiling_on_sc=False),
    )
    def sc_pool(tok_hbm, emb_hbm, pooled_hbm, idx_v, rows_v, pooled_v, sem):
        wid = lax.axis_index("s") * NUM_CORES + lax.axis_index("c")

        def chunk_body(c, carry):
            g = wid * chunks_per_w + c
            pltpu.sync_copy(tok_hbm.at[g], idx_v)
            copies = [
                pltpu.async_copy(emb_hbm.at[idx_v.at[j]],
                                 rows_v.at[pl.ds(j * gb, gb)], sem)
                for j in range(nb)
            ]
            for cp in copies:
                cp.wait()
            for b in range(cb):
                base = b * seq

                def tok_body(t, accs):
                    r = base + t
                    return tuple(
                        accs[d] + rows_v[r, pl.ds(d * LANES, LANES)]
                        for d in range(dgroups))

                accs = lax.fori_loop(
                    0, seq, tok_body,
                    tuple(jnp.zeros((LANES,), jnp.float32)
                          for _ in range(dgroups)))
                for d in range(dgroups):
                    pooled_v[b, pl.ds(d * LANES, LANES)] = accs[d] * inv
            pltpu.sync_copy(pooled_v, pooled_hbm.at[pl.ds(g * cb, cb)])
            return carry

        lax.fori_loop(0, chunks_per_w, chunk_body, 0)

    return sc_pool


def _mm_body(x_ref, w_ref, b_ref, o_ref):
    o_ref[...] = (jnp.dot(x_ref[...], w_ref[...],
                          preferred_element_type=jnp.float32)
                  + b_ref[...])


def kernel(token_ids, embedding, W, b):
    batch, seq = token_ids.shape
    vocab, emb_dim = embedding.shape
    out_dim = W.shape[1]

    cb = 16
    gb = 100
    tok_per_chunk = cb * seq
    nb = tok_per_chunk // gb
    total_chunks = batch // cb

    tok = token_ids.astype(jnp.int32).reshape(total_chunks, nb, gb)
    sc_pool = _make_sc_pool(batch, seq, emb_dim, cb, gb)
    pooled = sc_pool(tok, embedding)

    bm = 512
    grid = batch // bm
    out = pl.pallas_call(
        _mm_body,
        grid=(grid,),
        in_specs=[
            pl.BlockSpec((bm, emb_dim), lambda i: (i, 0)),
            pl.BlockSpec((emb_dim, out_dim), lambda i: (0, 0)),
            pl.BlockSpec((1, out_dim), lambda i: (0, 0)),
        ],
        out_specs=pl.BlockSpec((bm, out_dim), lambda i: (i, 0)),
        out_shape=jax.ShapeDtypeStruct((batch, out_dim), jnp.float32),
    )(pooled, W, b.reshape(1, out_dim))
    return out


# speedup vs baseline: 1.9811x; 1.9811x over previous
import functools

import jax
import jax.numpy as jnp
from jax import lax
from jax.experimental import pallas as pl
from jax.experimental.pallas import tpu as pltpu
from jax.experimental.pallas import tpu_sc as plsc

NUM_CORES = 2
NUM_SUBCORES = 16
NW = NUM_CORES * NUM_SUBCORES
LANES = 16
GB = 128


def _make_sc_pool(batch, seq, emb_dim, pad_dim, cb):
    rows_per_w = batch // NW
    chunks_per_w = rows_per_w // cb
    tok_real = cb * seq
    full_bursts = (tok_real + GB - 1) // GB
    slots = 8 * GB
    dgroups = emb_dim // LANES
    inv = 1.0 / float(seq)

    mesh = plsc.VectorSubcoreMesh(
        core_axis_name="c", subcore_axis_name="s",
        num_cores=NUM_CORES, num_subcores=NUM_SUBCORES)

    @functools.partial(
        pl.kernel,
        out_type=jax.ShapeDtypeStruct((batch, emb_dim), jnp.float32),
        mesh=mesh,
        scratch_types=[
            pltpu.VMEM((8, GB), jnp.int32),
            pltpu.VMEM((full_bursts * GB, pad_dim), jnp.float32),
            pltpu.VMEM((cb, emb_dim), jnp.float32),
            pltpu.SemaphoreType.DMA,
        ],
    )
    def sc_pool(tok_hbm, emb_hbm, pooled_hbm, idx_v, rows_v, pooled_v, sem):
        wid = lax.axis_index("s") * NUM_CORES + lax.axis_index("c")

        def chunk_body(c, carry):
            g = wid * chunks_per_w + c
            pltpu.sync_copy(tok_hbm.at[g], idx_v)
            copies = [
                pltpu.async_copy(emb_hbm.at[idx_v.at[j]],
                                 rows_v.at[pl.ds(j * GB, GB)], sem)
                for j in range(full_bursts)
            ]
            for cp in copies:
                cp.wait()
            for b in range(cb):
                base = b * seq

                def tok_body(t, accs):
                    r = base + t
                    return tuple(
                        accs[d] + rows_v[r, pl.ds(d * LANES, LANES)]
                        for d in range(dgroups))

                accs = lax.fori_loop(
                    0, seq, tok_body,
                    tuple(jnp.zeros((LANES,), jnp.float32)
                          for _ in range(dgroups)))
                for d in range(dgroups):
                    pooled_v[b, pl.ds(d * LANES, LANES)] = accs[d] * inv
            pltpu.sync_copy(pooled_v, pooled_hbm.at[pl.ds(g * cb, cb)])
            return carry

        lax.fori_loop(0, chunks_per_w, chunk_body, 0)

    return sc_pool


def _mm_body(x_ref, w_ref, b_ref, o_ref):
    o_ref[...] = (jnp.dot(x_ref[...], w_ref[...],
                          preferred_element_type=jnp.float32)
                  + b_ref[...])


def kernel(token_ids, embedding, W, b):
    batch, seq = token_ids.shape
    vocab, emb_dim = embedding.shape
    out_dim = W.shape[1]
    pad_dim = 128

    cb = 16
    tok_real = cb * seq
    slots = 8 * GB
    total_chunks = batch // cb

    eye_pad = jnp.eye(emb_dim, pad_dim, dtype=jnp.float32)
    emb_pad = embedding @ eye_pad

    tok_flat = token_ids.astype(jnp.int32).reshape(total_chunks, tok_real)
    n_pad = slots - tok_real
    pads = jnp.broadcast_to(
        (jnp.arange(n_pad, dtype=jnp.int32) * 4099) % vocab,
        (total_chunks, n_pad))
    tok3 = jnp.concatenate([tok_flat, pads], axis=1).reshape(
        total_chunks, 8, GB)

    sc_pool = _make_sc_pool(batch, seq, emb_dim, pad_dim, cb)
    pooled = sc_pool(tok3, emb_pad)

    bm = 512
    grid = batch // bm
    out = pl.pallas_call(
        _mm_body,
        grid=(grid,),
        in_specs=[
            pl.BlockSpec((bm, emb_dim), lambda i: (i, 0)),
            pl.BlockSpec((emb_dim, out_dim), lambda i: (0, 0)),
            pl.BlockSpec((1, out_dim), lambda i: (0, 0)),
        ],
        out_specs=pl.BlockSpec((bm, out_dim), lambda i: (i, 0)),
        out_shape=jax.ShapeDtypeStruct((batch, out_dim), jnp.float32),
    )(pooled, W, b.reshape(1, out_dim))
    return out


# speedup vs baseline: 2.0763x; 1.0481x over previous
import functools

import jax
import jax.numpy as jnp
from jax import lax
from jax.experimental import pallas as pl
from jax.experimental.pallas import tpu as pltpu
from jax.experimental.pallas import tpu_sc as plsc

NUM_CORES = 2
NUM_SUBCORES = 16
NW = NUM_CORES * NUM_SUBCORES
LANES = 16
GB = 128


def _make_sc_pool(batch, seq, emb_dim, pad_dim, cb):
    rows_per_w = batch // NW
    chunks_per_w = rows_per_w // cb
    tok_real = cb * seq
    full_bursts = (tok_real + GB - 1) // GB
    slots = 8 * GB
    dgroups = emb_dim // LANES
    inv = 1.0 / float(seq)

    mesh = plsc.VectorSubcoreMesh(
        core_axis_name="c", subcore_axis_name="s",
        num_cores=NUM_CORES, num_subcores=NUM_SUBCORES)

    @functools.partial(
        pl.kernel,
        out_type=jax.ShapeDtypeStruct((batch, emb_dim), jnp.float32),
        mesh=mesh,
        scratch_types=[
            pltpu.VMEM((8, GB), jnp.int32),
            pltpu.VMEM((full_bursts * GB, pad_dim), jnp.float32),
            pltpu.VMEM((cb, emb_dim), jnp.float32),
            pltpu.SemaphoreType.DMA,
        ],
    )
    def sc_pool(tok_hbm, emb_hbm, pooled_hbm, idx_v, rows_v, pooled_v, sem):
        wid = lax.axis_index("s") * NUM_CORES + lax.axis_index("c")

        def accum_row(b):
            base = b * seq

            def tok_body(t, accs):
                r = base + t
                return tuple(
                    accs[d] + rows_v[r, pl.ds(d * LANES, LANES)]
                    for d in range(dgroups))

            accs = lax.fori_loop(
                0, seq, tok_body,
                tuple(jnp.zeros((LANES,), jnp.float32)
                      for _ in range(dgroups)),
                unroll=5)
            for d in range(dgroups):
                pooled_v[b, pl.ds(d * LANES, LANES)] = accs[d] * inv

        def chunk_body(c, carry):
            g = wid * chunks_per_w + c
            pltpu.sync_copy(tok_hbm.at[g], idx_v)
            copies = [
                pltpu.async_copy(emb_hbm.at[idx_v.at[j]],
                                 rows_v.at[pl.ds(j * GB, GB)], sem)
                for j in range(full_bursts)
            ]
            done = 0
            for j in range(full_bursts):
                copies[j].wait()
                hi = min(cb, (GB * (j + 1) - seq) // seq + 1)
                for b in range(done, hi):
                    accum_row(b)
                done = hi
            for b in range(done, cb):
                accum_row(b)
            pltpu.sync_copy(pooled_v, pooled_hbm.at[pl.ds(g * cb, cb)])
            return carry

        lax.fori_loop(0, chunks_per_w, chunk_body, 0)

    return sc_pool


def _mm_body(x_ref, w_ref, b_ref, o_ref):
    o_ref[...] = (jnp.dot(x_ref[...], w_ref[...],
                          preferred_element_type=jnp.float32)
                  + b_ref[...])


def kernel(token_ids, embedding, W, b):
    batch, seq = token_ids.shape
    vocab, emb_dim = embedding.shape
    out_dim = W.shape[1]
    pad_dim = 128

    cb = 16
    tok_real = cb * seq
    slots = 8 * GB
    total_chunks = batch // cb

    eye_pad = jnp.eye(emb_dim, pad_dim, dtype=jnp.float32)
    emb_pad = embedding @ eye_pad

    tok_flat = token_ids.astype(jnp.int32).reshape(total_chunks, tok_real)
    n_pad = slots - tok_real
    pads = jnp.broadcast_to(
        (jnp.arange(n_pad, dtype=jnp.int32) * 4099) % vocab,
        (total_chunks, n_pad))
    tok3 = jnp.concatenate([tok_flat, pads], axis=1).reshape(
        total_chunks, 8, GB)

    sc_pool = _make_sc_pool(batch, seq, emb_dim, pad_dim, cb)
    pooled = sc_pool(tok3, emb_pad)

    bm = 512
    grid = batch // bm
    out = pl.pallas_call(
        _mm_body,
        grid=(grid,),
        in_specs=[
            pl.BlockSpec((bm, emb_dim), lambda i: (i, 0)),
            pl.BlockSpec((emb_dim, out_dim), lambda i: (0, 0)),
            pl.BlockSpec((1, out_dim), lambda i: (0, 0)),
        ],
        out_specs=pl.BlockSpec((bm, out_dim), lambda i: (i, 0)),
        out_shape=jax.ShapeDtypeStruct((batch, out_dim), jnp.float32),
    )(pooled, W, b.reshape(1, out_dim))
    return out
